# baseline (speedup 1.0000x reference)
import functools

import jax
import jax.numpy as jnp
import numpy as np
from jax import lax
from jax.experimental import pallas as pl
from jax.experimental.pallas import tpu as pltpu
from jax.experimental.pallas import tpu_sc as plsc

NU = 50000
NI = 50000
D = 32
N = NU + NI
E = 1600000
B = 4096
NLAYERS = 3

NC = 2
NS = 16
LANES = 16
HALF = D // NC
CH = 256
NB = 6
NCHT = -(-E // (NS * CH * NB)) * NB
EPT = NCHT * CH
EPAD = EPT * NS
NP = 100096
RPT = NP // NS
ZR = 3128
BPT = B // NS


def _sc_body(emb, colp, rowp, valp, bidx, zeros, usum, psum, nsum,
             l1, l2, l3, acc, *scratch):
    c = lax.axis_index("c")
    s = lax.axis_index("s")
    tile_e0 = s * EPT

    bufs = tuple(scratch[7 * b:7 * b + 7] for b in range(NB))
    semg = scratch[7 * NB]
    srcs = (emb.at[c], l1.at[c], l2.at[c], l3.at[c])
    dsts = (l1, l2, l3)

    def issue_lin(slot, cidx):
        base = tile_e0 + cidx * CH
        colr, rowr, valr, _, semL, _, _ = bufs[slot]
        pltpu.async_copy(colp.at[pl.ds(base, CH)], colr, semL)
        pltpu.async_copy(rowp.at[pl.ds(base, CH)], rowr, semL)
        pltpu.async_copy(valp.at[pl.ds(base, CH)], valr, semL)

    def wait_lin(slot):
        colr, rowr, valr, _, semL, _, _ = bufs[slot]
        pltpu.make_async_copy(colp.at[pl.ds(0, CH)], colr, semL).wait()
        pltpu.make_async_copy(rowp.at[pl.ds(0, CH)], rowr, semL).wait()
        pltpu.make_async_copy(valp.at[pl.ds(0, CH)], valr, semL).wait()

    def scale_chunk(slot):
        _, _, valr, rowsr, _, _, _ = bufs[slot]

        @pl.loop(0, CH // LANES)
        def _scale(g):
            vvec = valr[pl.ds(g * LANES, LANES)]
            for e0 in range(LANES):
                e = g * LANES + e0
                rowsr[e] = rowsr[e] * vvec[e0]

    for l in range(NLAYERS):
        src = srcs[l]
        dst = dsts[l]
        for k in range(RPT // ZR):
            pltpu.sync_copy(zeros, acc.at[pl.ds(s * RPT + k * ZR, ZR)])
        plsc.subcore_barrier()

        for b in range(NB):
            issue_lin(b, b)

        @pl.loop(0, NCHT, step=NB)
        def _edge_group(g0):
            for b in range(NB):
                colr, rowr, valr, rowsr, semL, semG, semS = bufs[b]
                wait_lin(b)
                pltpu.async_copy(src.at[colr], rowsr, semG)
            for b in range(NB):
                colr, rowr, valr, rowsr, semL, semG, semS = bufs[b]
                pltpu.make_async_copy(src.at[colr], rowsr, semG).wait()
                scale_chunk(b)
                pltpu.async_copy(rowsr, acc.at[rowr], semS, add=True)
            for b in range(NB):
                colr, rowr, valr, rowsr, semL, semG, semS = bufs[b]
                pltpu.make_async_copy(rowsr, acc.at[rowr], semS).wait()

                @pl.when(g0 + NB + b < NCHT)
                def _():
                    issue_lin(b, g0 + NB + b)

        plsc.subcore_barrier()
        for k in range(RPT // ZR):
            r0 = s * RPT + k * ZR
            pltpu.sync_copy(acc.at[pl.ds(r0, ZR)], dst.at[c, pl.ds(r0, ZR)])
        plsc.subcore_barrier()

    stages = tuple(bufs[b][3] for b in range(4))
    idxb = bufs[0][0]
    outs = (usum, psum, nsum)
    for cc in range(BPT // CH):
        b0 = s * BPT + cc * CH
        for t in range(3):
            pltpu.sync_copy(bidx.at[pl.ds(t * B + b0, CH)], idxb)
            cps = [pltpu.async_copy(srcs[l].at[idxb], stages[l], semg)
                   for l in range(4)]
            for cp in cps:
                cp.wait()

            @pl.loop(0, CH, unroll=8)
            def _sum_rows(e):
                stages[0][e] = ((stages[0][e] + stages[1][e]) +
                                (stages[2][e] + stages[3][e]))

            pltpu.sync_copy(stages[0], outs[t].at[c, pl.ds(b0, CH)])


_SLOT_SCRATCH = []
for _ in range(NB):
    _SLOT_SCRATCH += [
        pltpu.VMEM((CH,), jnp.int32),
        pltpu.VMEM((CH,), jnp.int32),
        pltpu.VMEM((CH,), jnp.float32),
        pltpu.VMEM((CH, HALF), jnp.float32),
        pltpu.SemaphoreType.DMA,
        pltpu.SemaphoreType.DMA,
        pltpu.SemaphoreType.DMA,
    ]


@functools.partial(
    pl.kernel,
    out_type=[
        jax.ShapeDtypeStruct((NC, B, HALF), jnp.float32),
        jax.ShapeDtypeStruct((NC, B, HALF), jnp.float32),
        jax.ShapeDtypeStruct((NC, B, HALF), jnp.float32),
        jax.ShapeDtypeStruct((NC, NP, HALF), jnp.float32),
        jax.ShapeDtypeStruct((NC, NP, HALF), jnp.float32),
        jax.ShapeDtypeStruct((NC, NP, HALF), jnp.float32),
    ],
    mesh=plsc.VectorSubcoreMesh(core_axis_name="c", subcore_axis_name="s"),
    compiler_params=pltpu.CompilerParams(use_tc_tiling_on_sc=False),
    scratch_types=[pltpu.VMEM_SHARED((NP, HALF), jnp.float32)]
    + _SLOT_SCRATCH
    + [pltpu.SemaphoreType.DMA],
)
def _sc_propagate(emb, colp, rowp, valp, bidx, zeros, usum, psum, nsum,
                  l1, l2, l3, *scratch):
    _sc_body(emb, colp, rowp, valp, bidx, zeros, usum, psum, nsum, l1, l2, l3,
             *scratch)


def _loss_body(u_ref, p_ref, n_ref, o_ref):
    u = u_ref[...]
    p = p_ref[...]
    n = n_ref[...]
    t = u * (p - n)
    rowx = jnp.sum(t, axis=1)
    x = (rowx[:B] + rowx[B:]) * (1.0 / 16.0)
    z = jnp.minimum(x, 0.0) - jnp.log(1.0 + jnp.exp(-jnp.abs(x)))
    o_ref[...] = jnp.reshape(-jnp.sum(z) * (1.0 / B), (1, 1))


_loss_call = pl.pallas_call(
    _loss_body,
    out_shape=jax.ShapeDtypeStruct((1, 1), jnp.float32),
)


def kernel(user_table, item_table, adj_val, adj_row, adj_col,
           user_idx, pos_item_idx, neg_item_idx):
    all_emb = jnp.concatenate([user_table, item_table], axis=0)
    halves = jnp.stack([all_emb[:, :HALF], all_emb[:, HALF:]])
    pad = EPAD - E
    colp = jnp.concatenate([adj_col, jnp.zeros((pad,), jnp.int32)])
    rowp = jnp.concatenate([adj_row, jnp.zeros((pad,), jnp.int32)])
    valp = jnp.concatenate([adj_val, jnp.zeros((pad,), jnp.float32)])
    bidx = jnp.concatenate([user_idx, pos_item_idx + NU, neg_item_idx + NU])
    zeros = jnp.zeros((ZR, HALF), jnp.float32)
    usum, psum, nsum, _, _, _ = _sc_propagate(halves, colp, rowp, valp, bidx,
                                              zeros)
    loss = _loss_call(jnp.reshape(usum, (NC * B, HALF)),
                      jnp.reshape(psum, (NC * B, HALF)),
                      jnp.reshape(nsum, (NC * B, HALF)))
    return loss[0, 0]


# speedup vs baseline: 1.2388x; 1.2388x over previous
import functools

import jax
import jax.numpy as jnp
import numpy as np
from jax import lax
from jax.experimental import pallas as pl
from jax.experimental.pallas import tpu as pltpu
from jax.experimental.pallas import tpu_sc as plsc

NU = 50000
NI = 50000
D = 32
N = NU + NI
E = 1600000
B = 4096
NLAYERS = 3

NC = 2
NS = 16
LANES = 16
HALF = D // NC
CH = 128
NB = 8
NCHT = -(-E // (NS * CH * NB)) * NB
EPT = NCHT * CH
EPAD = EPT * NS
NP = 100096
RPT = NP // NS
ZR = 3128
BPT = B // NS


def _sc_body(emb, edata, bidx, zeros, usum, psum, nsum,
             l1, l2, l3, acc, *scratch):
    c = lax.axis_index("c")
    s = lax.axis_index("s")
    tile_e0 = s * EPT

    bufs = tuple(scratch[5 * b:5 * b + 5] for b in range(NB))
    semg = scratch[5 * NB]
    srcs = (emb.at[c], l1.at[c], l2.at[c], l3.at[c])
    dsts = (l1, l2, l3)

    def issue_lin(slot, cidx):
        ebuf, _, semL, _, _ = bufs[slot]
        pltpu.async_copy(edata.at[s * NCHT + cidx], ebuf, semL)

    def wait_lin(slot):
        ebuf, _, semL, _, _ = bufs[slot]
        pltpu.make_async_copy(edata.at[0], ebuf, semL).wait()

    def scale_chunk(slot):
        ebuf, rowsr, _, _, _ = bufs[slot]

        @pl.loop(0, CH // LANES)
        def _scale(g):
            vvec = lax.bitcast_convert_type(ebuf[2, pl.ds(g * LANES, LANES)], jnp.float32)
            for e0 in range(LANES):
                e = g * LANES + e0
                rowsr[e] = rowsr[e] * vvec[e0]

    for l in range(NLAYERS):
        src = srcs[l]
        dst = dsts[l]
        for k in range(RPT // ZR):
            pltpu.sync_copy(zeros, acc.at[pl.ds(s * RPT + k * ZR, ZR)])
        plsc.subcore_barrier()

        for b in range(NB):
            issue_lin(b, b)

        @pl.loop(0, NCHT, step=NB)
        def _edge_group(g0):
            for b in range(NB):
                ebuf, rowsr, semL, semG, semS = bufs[b]
                wait_lin(b)
                pltpu.async_copy(src.at[ebuf.at[0]], rowsr, semG)
            for b in range(NB):
                ebuf, rowsr, semL, semG, semS = bufs[b]
                pltpu.make_async_copy(src.at[ebuf.at[0]], rowsr, semG).wait()
                scale_chunk(b)
                pltpu.async_copy(rowsr, acc.at[ebuf.at[1]], semS, add=True)
            for b in range(NB):
                ebuf, rowsr, semL, semG, semS = bufs[b]
                pltpu.make_async_copy(rowsr, acc.at[ebuf.at[1]], semS).wait()

                @pl.when(g0 + NB + b < NCHT)
                def _():
                    issue_lin(b, g0 + NB + b)

        plsc.subcore_barrier()
        for k in range(RPT // ZR):
            r0 = s * RPT + k * ZR
            pltpu.sync_copy(acc.at[pl.ds(r0, ZR)], dst.at[c, pl.ds(r0, ZR)])
        plsc.subcore_barrier()

    stages = tuple(bufs[b][1] for b in range(4))
    idxb = bufs[0][0].at[0]
    outs = (usum, psum, nsum)
    for cc in range(BPT // CH):
        b0 = s * BPT + cc * CH
        for t in range(3):
            pltpu.sync_copy(bidx.at[pl.ds(t * B + b0, CH)], idxb)
            cps = [pltpu.async_copy(srcs[l].at[idxb], stages[l], semg)
                   for l in range(4)]
            for cp in cps:
                cp.wait()

            @pl.loop(0, CH, unroll=8)
            def _sum_rows(e):
                stages[0][e] = ((stages[0][e] + stages[1][e]) +
                                (stages[2][e] + stages[3][e]))

            pltpu.sync_copy(stages[0], outs[t].at[c, pl.ds(b0, CH)])


_SLOT_SCRATCH = []
for _ in range(NB):
    _SLOT_SCRATCH += [
        pltpu.VMEM((3, CH), jnp.int32),
        pltpu.VMEM((CH, HALF), jnp.float32),
        pltpu.SemaphoreType.DMA,
        pltpu.SemaphoreType.DMA,
        pltpu.SemaphoreType.DMA,
    ]


@functools.partial(
    pl.kernel,
    out_type=[
        jax.ShapeDtypeStruct((NC, B, HALF), jnp.float32),
        jax.ShapeDtypeStruct((NC, B, HALF), jnp.float32),
        jax.ShapeDtypeStruct((NC, B, HALF), jnp.float32),
        jax.ShapeDtypeStruct((NC, NP, HALF), jnp.float32),
        jax.ShapeDtypeStruct((NC, NP, HALF), jnp.float32),
        jax.ShapeDtypeStruct((NC, NP, HALF), jnp.float32),
    ],
    mesh=plsc.VectorSubcoreMesh(core_axis_name="c", subcore_axis_name="s"),
    compiler_params=pltpu.CompilerParams(use_tc_tiling_on_sc=False),
    scratch_types=[pltpu.VMEM_SHARED((NP, HALF), jnp.float32)]
    + _SLOT_SCRATCH
    + [pltpu.SemaphoreType.DMA],
)
def _sc_propagate(emb, edata, bidx, zeros, usum, psum, nsum,
                  l1, l2, l3, *scratch):
    _sc_body(emb, edata, bidx, zeros, usum, psum, nsum, l1, l2, l3,
             *scratch)


def _loss_body(u_ref, p_ref, n_ref, o_ref):
    u = u_ref[...]
    p = p_ref[...]
    n = n_ref[...]
    t = u * (p - n)
    rowx = jnp.sum(t, axis=1)
    x = (rowx[:B] + rowx[B:]) * (1.0 / 16.0)
    z = jnp.minimum(x, 0.0) - jnp.log(1.0 + jnp.exp(-jnp.abs(x)))
    o_ref[...] = jnp.reshape(-jnp.sum(z) * (1.0 / B), (1, 1))


_loss_call = pl.pallas_call(
    _loss_body,
    out_shape=jax.ShapeDtypeStruct((1, 1), jnp.float32),
)


def kernel(user_table, item_table, adj_val, adj_row, adj_col,
           user_idx, pos_item_idx, neg_item_idx):
    all_emb = jnp.concatenate([user_table, item_table], axis=0)
    halves = jnp.stack([all_emb[:, :HALF], all_emb[:, HALF:]])
    pad = EPAD - E
    spread = jnp.arange(pad, dtype=jnp.int32) % N
    colp = jnp.concatenate([adj_col, spread])
    rowp = jnp.concatenate([adj_row, spread])
    valp = jnp.concatenate([adj_val, jnp.zeros((pad,), jnp.float32)])
    edata = jnp.stack([jnp.reshape(colp, (-1, CH)),
                       jnp.reshape(rowp, (-1, CH)),
                       jnp.reshape(valp, (-1, CH)).view(jnp.int32)], axis=1)
    bidx = jnp.concatenate([user_idx, pos_item_idx + NU, neg_item_idx + NU])
    zeros = jnp.zeros((ZR, HALF), jnp.float32)
    usum, psum, nsum, _, _, _ = _sc_propagate(halves, edata, bidx, zeros)
    loss = _loss_call(jnp.reshape(usum, (NC * B, HALF)),
                      jnp.reshape(psum, (NC * B, HALF)),
                      jnp.reshape(nsum, (NC * B, HALF)))
    return loss[0, 0]
